# baseline (speedup 1.0000x reference)
import jax
import jax.numpy as jnp
from jax.experimental import pallas as pl
from jax.experimental.pallas import tpu as pltpu


def kernel(x, pe_weight):
    seq_len = x.shape[1]
    n_rows, dim = pe_weight.shape
    del n_rows

    block_rows = 1024
    while seq_len % block_rows:
        block_rows //= 2

    def copy_body(src_ref, out_ref):
        out_ref[...] = src_ref[...]

    out = pl.pallas_call(
        copy_body,
        grid=(seq_len // block_rows,),
        out_shape=jax.ShapeDtypeStruct((seq_len, dim), pe_weight.dtype),
        in_specs=[pl.BlockSpec((block_rows, dim), lambda i: (i, 0))],
        out_specs=pl.BlockSpec((block_rows, dim), lambda i: (i, 0)),
        compiler_params=pltpu.CompilerParams(
            dimension_semantics=("parallel",),
        ),
    )(pe_weight)
    return out[None, ...]
